# Initial kernel scaffold; baseline (speedup 1.0000x reference)
import jax
import jax.numpy as jnp
from jax.experimental import pallas as pl


def kernel(x, table):
    raise NotImplementedError("write your pallas kernel here")


# speedup vs baseline: 3.3420x; 3.3420x over previous
import functools

import jax
import jax.numpy as jnp
from jax import lax
from jax.experimental import pallas as pl
from jax.experimental.pallas import tpu as pltpu
from jax.experimental.pallas import tpu_sc as plsc

NC = 2
NS = 16
NW = NC * NS

CHUNK = 128


def _body(nchunks, x_hbm, table_hbm, out_hbm, idx_v, rows0, rows1, sem0, sem1):
  wid = lax.axis_index("s") * NC + lax.axis_index("c")
  base = wid * nchunks
  pltpu.sync_copy(x_hbm.at[wid], idx_v)

  rows = (rows0, rows1)
  sems = (sem0, sem1)

  pltpu.async_copy(table_hbm.at[idx_v.at[0]], rows0, sem0)

  def step(j, carry):
    slot = lax.rem(j, 2)

    @pl.when(j + 1 < nchunks)
    def _():
      nslot = lax.rem(j + 1, 2)
      for b in range(2):
        @pl.when(nslot == b)
        def _():
          pltpu.async_copy(table_hbm.at[idx_v.at[j + 1]], rows[b], sems[b])

    for b in range(2):
      @pl.when(slot == b)
      def _():
        pltpu.make_async_copy(table_hbm.at[idx_v.at[j]], rows[b], sems[b]).wait()
        pltpu.sync_copy(rows[b], out_hbm.at[base + j])
    return carry

  lax.fori_loop(0, nchunks, step, 0)


def kernel(x, table):
  B, S = x.shape
  V, D = table.shape
  n = B * S
  assert n % (NW * CHUNK) == 0 and D == CHUNK
  nchunks = n // (NW * CHUNK)

  x2 = x.reshape(NW, nchunks, CHUNK).astype(jnp.int32)

  mesh = plsc.VectorSubcoreMesh(core_axis_name="c", subcore_axis_name="s")
  k = pl.kernel(
      functools.partial(_body, nchunks),
      out_type=jax.ShapeDtypeStruct((n // CHUNK, CHUNK, D), jnp.float32),
      mesh=mesh,
      scratch_types=[
          pltpu.VMEM((nchunks, CHUNK), jnp.int32),
          pltpu.VMEM((CHUNK, D), jnp.float32),
          pltpu.VMEM((CHUNK, D), jnp.float32),
          pltpu.SemaphoreType.DMA,
          pltpu.SemaphoreType.DMA,
      ],
  )
  out = k(x2, table)
  return out.reshape(B, S, D)
